# Initial kernel scaffold; baseline (speedup 1.0000x reference)
import jax
import jax.numpy as jnp
from jax.experimental import pallas as pl


def kernel(nodes, dist, fied, emb, Ws, As):
    raise NotImplementedError("write your pallas kernel here")


# speedup vs baseline: 3.0036x; 3.0036x over previous
import functools

import jax
import jax.numpy as jnp
from jax import lax
from jax.experimental import pallas as pl
from jax.experimental.pallas import tpu as pltpu
from jax.experimental.pallas import tpu_sc as plsc

DIM = 256
N = 2048
NHEADS = 8
HD = DIM // NHEADS
ALPHA = 0.2
BR = 256

_NC = 2
_NS = 16
_NW = _NC * _NS
_BPW = N // _NW


def _gather_sc(emb, nodes):
    mesh = plsc.VectorSubcoreMesh(core_axis_name="c", subcore_axis_name="s")

    @functools.partial(
        pl.kernel,
        mesh=mesh,
        out_type=jax.ShapeDtypeStruct((N, DIM), jnp.float32),
        scratch_types=[
            pltpu.VMEM((_BPW,), jnp.int32),
            pltpu.VMEM((_BPW, DIM), jnp.float32),
            pltpu.SemaphoreType.DMA,
        ],
    )
    def k(emb_hbm, idx_hbm, out_hbm, idx_v, rows_v, sem):
        wid = lax.axis_index("s") * _NC + lax.axis_index("c")
        base = wid * _BPW
        pltpu.sync_copy(idx_hbm.at[pl.ds(base, _BPW)], idx_v)
        pltpu.async_copy(emb_hbm.at[idx_v], rows_v, sem).wait()
        pltpu.sync_copy(rows_v, out_hbm.at[pl.ds(base, _BPW)])

    return k(emb, nodes)


def _phase_a_body(h_ref, wcat_ref, amat_ref, whb_ref, us_ref, vt_ref):
    hv = h_ref[...]
    wh = jnp.dot(hv, wcat_ref[...], preferred_element_type=jnp.float32)
    whb_ref[...] = wh.astype(jnp.bfloat16)
    e = jnp.dot(wh, amat_ref[...], preferred_element_type=jnp.float32)
    es = e[:, :NHEADS]
    ed = e[:, NHEADS:]
    ms = jnp.max(es, axis=0, keepdims=True)
    md = jnp.max(ed, axis=0, keepdims=True)
    m = ms + md
    u1 = jnp.exp(es - ms)
    ua = jnp.exp(ALPHA * (es - ms))
    v1 = jnp.exp(ed - md)
    va = jnp.exp(ALPHA * (ed - md) - (1.0 - ALPHA) * m)
    us_ref[...] = jnp.concatenate([u1, ua], axis=1)
    vt_ref[...] = jnp.concatenate([v1, va], axis=1).T


def _phase_a(h, wcat, amat):
    return pl.pallas_call(
        _phase_a_body,
        out_shape=(
            jax.ShapeDtypeStruct((N, DIM), jnp.bfloat16),
            jax.ShapeDtypeStruct((N, 2 * NHEADS), jnp.float32),
            jax.ShapeDtypeStruct((2 * NHEADS, N), jnp.float32),
        ),
    )(h, wcat, amat)


def _phase_b_body(dist_ref, whb_ref, us_ref, vt_ref, out_ref):
    adjf = jnp.where(dist_ref[...] > 0.5, 1.0, 0.0).astype(jnp.float32)
    us = us_ref[...]
    vt = vt_ref[...]
    whb = whb_ref[...]
    for h in range(NHEADS):
        u1 = us[:, h:h + 1]
        ua = us[:, NHEADS + h:NHEADS + h + 1]
        v1 = vt[h:h + 1, :]
        va = vt[NHEADS + h:NHEADS + h + 1, :]
        p = adjf * jnp.maximum(u1 * v1, ua * va)
        rs = jnp.sum(p, axis=1, keepdims=True)
        ob = jnp.dot(
            p.astype(jnp.bfloat16),
            whb[:, h * HD:(h + 1) * HD],
            preferred_element_type=jnp.float32,
        )
        o = ob / rs
        out_ref[:, h * HD:(h + 1) * HD] = jnp.where(o > 0, o, jnp.exp(o) - 1.0)


def _phase_b(dist, whb, us, vt):
    return pl.pallas_call(
        _phase_b_body,
        grid=(N // BR,),
        in_specs=[
            pl.BlockSpec((BR, N), lambda i: (i, 0)),
            pl.BlockSpec((N, DIM), lambda i: (0, 0)),
            pl.BlockSpec((BR, 2 * NHEADS), lambda i: (i, 0)),
            pl.BlockSpec((2 * NHEADS, N), lambda i: (0, 0)),
        ],
        out_specs=pl.BlockSpec((BR, DIM), lambda i: (i, 0)),
        out_shape=jax.ShapeDtypeStruct((N, DIM), jnp.float32),
    )(dist, whb, us, vt)


def _gat_tc(h, dist, Ws, As):
    n_units = Ws.shape[0]
    eye = jnp.eye(NHEADS, dtype=jnp.float32)
    for u in range(n_units):
        wcat = Ws[u].transpose(1, 0, 2).reshape(DIM, DIM)
        a_src = As[u, :, :HD, 0]
        a_dst = As[u, :, HD:, 0]
        asrc_m = (eye[:, None, :] * a_src[:, :, None]).reshape(DIM, NHEADS)
        adst_m = (eye[:, None, :] * a_dst[:, :, None]).reshape(DIM, NHEADS)
        amat = jnp.concatenate([asrc_m, adst_m], axis=1)
        whb, us, vt = _phase_a(h, wcat, amat)
        h = _phase_b(dist, whb, us, vt)
    return h


def kernel(nodes, dist, fied, emb, Ws, As):
    h = _gather_sc(emb, nodes)
    return _gat_tc(h, dist, Ws, As)
